# Initial kernel scaffold; baseline (speedup 1.0000x reference)
import jax
import jax.numpy as jnp
from jax.experimental import pallas as pl


def kernel(feat0, feat1, adj, emb, enc_W, enc_b, W_ac, Wh0, av0, Wb0, bb0, qb0, fcW0, fcb0, Wh1, av1, Wb1, bb1, qb1, fcW1, fcb1, type_mask, feat_keep_idx, feat_drop_idx, mp0_idx, mp1_idx, target_node_indices, epoch, flag):
    raise NotImplementedError("write your pallas kernel here")


# speedup vs baseline: 3.9966x; 3.9966x over previous
import functools

import jax
import jax.numpy as jnp
from jax.experimental import pallas as pl

N0, N1, DIN, DH, HEADS, DEMB, ATT, OUT = 4000, 2000, 128, 128, 4, 64, 64, 8
HD = HEADS * DH


def _fc_kernel(x_ref, w_ref, b_ref, o_ref, *, act):
    y = jnp.dot(x_ref[:], w_ref[:], preferred_element_type=jnp.float32) + b_ref[:]
    if act == 'elu':
        y = jnp.where(y > 0, y, jnp.exp(y) - 1.0)
    o_ref[:] = y


def _fc(x, w, b, act, mb):
    m, din = x.shape
    dout = w.shape[1]
    return pl.pallas_call(
        functools.partial(_fc_kernel, act=act),
        grid=(m // mb,),
        in_specs=[pl.BlockSpec((mb, din), lambda i: (i, 0)),
                  pl.BlockSpec((din, dout), lambda i: (0, 0)),
                  pl.BlockSpec((1, dout), lambda i: (0, 0))],
        out_specs=pl.BlockSpec((mb, dout), lambda i: (i, 0)),
        out_shape=jax.ShapeDtypeStruct((m, dout), jnp.float32),
    )(x, w, b.reshape(1, -1))


def _att_kernel(ed_ref, esT_ref, fs_ref, adj_ref, w_ref, o_ref):
    ed = ed_ref[:]
    adj = adj_ref[:]
    esT = esT_ref[:]
    fs = fs_ref[:]
    acc = None
    for h in range(HEADS):
        w = w_ref[h]
        hd = jnp.dot(ed, w, preferred_element_type=jnp.float32)
        hd2 = jax.lax.dot_general(hd, w, (((1,), (1,)), ((), ())),
                                  preferred_element_type=jnp.float32)
        e = jnp.dot(hd2, esT, preferred_element_type=jnp.float32)
        e = jnp.where(e >= 0, e, 0.2 * e)
        e = jnp.where(adj > 0.5, e, -1e9)
        m = jnp.max(e, axis=1, keepdims=True)
        ex = jnp.exp(e - m)
        a = ex / jnp.sum(ex, axis=1, keepdims=True)
        t = jnp.dot(a, fs, preferred_element_type=jnp.float32)
        acc = t if acc is None else acc + t
    r = acc * (1.0 / HEADS)
    o_ref[:] = jnp.where(r > 0, r, jnp.exp(r) - 1.0)


def _att(ed, esT, fs, adj, w_ac, mb):
    m = ed.shape[0]
    kp = esT.shape[1]
    return pl.pallas_call(
        _att_kernel,
        grid=(m // mb,),
        in_specs=[pl.BlockSpec((mb, DEMB), lambda i: (i, 0)),
                  pl.BlockSpec((DEMB, kp), lambda i: (0, 0)),
                  pl.BlockSpec((kp, DH), lambda i: (0, 0)),
                  pl.BlockSpec((mb, kp), lambda i: (i, 0)),
                  pl.BlockSpec((HEADS, DEMB, DEMB), lambda i: (0, 0, 0))],
        out_specs=pl.BlockSpec((mb, DH), lambda i: (i, 0)),
        out_shape=jax.ShapeDtypeStruct((m, DH), jnp.float32),
    )(ed, esT, fs, adj, w_ac)


def _edge_kernel(h_ref, wh_ref, wv_ref, eft_ref, e_ref):
    h = h_ref[:]
    eft_ref[:] = jnp.dot(h, wh_ref[:], preferred_element_type=jnp.float32)
    z = jnp.dot(h, wv_ref[:], preferred_element_type=jnp.float32)
    e_ref[:] = jnp.where(z >= 0, z, 0.2 * z)


def _edge(h_inst, wh, wv, eb):
    e_n = h_inst.shape[0]
    return pl.pallas_call(
        _edge_kernel,
        grid=(e_n // eb,),
        in_specs=[pl.BlockSpec((eb, DH), lambda i: (i, 0)),
                  pl.BlockSpec((DH, HD), lambda i: (0, 0)),
                  pl.BlockSpec((DH, HEADS), lambda i: (0, 0))],
        out_specs=[pl.BlockSpec((eb, HD), lambda i: (i, 0)),
                   pl.BlockSpec((eb, HEADS), lambda i: (i, 0))],
        out_shape=[jax.ShapeDtypeStruct((e_n, HD), jnp.float32),
                   jax.ShapeDtypeStruct((e_n, HEADS), jnp.float32)],
    )(h_inst, wh, wv)


def _layer(tf, mp0, mp1, Wh, av, Wb, bb, qb):
    his = []
    beta_last = None
    for i, (mp, ni, off) in enumerate([(mp0, N0, 0), (mp1, N1, N0)]):
        part = jax.lax.dynamic_slice_in_dim(tf, off, ni, 0)
        outs = []
        for p in range(2):
            idx = mp[p]
            dst = idx[:, 0]
            en = idx.shape[0]
            g = jnp.take(part, idx.reshape(-1), axis=0).reshape(en, 3, DH)
            h_inst = jnp.mean(g, axis=1)
            wv = jnp.einsum('dhk,hk->dh', Wh[i, p].reshape(DH, HEADS, DH), av[i, p])
            eft, e = _edge(h_inst, Wh[i, p], wv, eb=2000)
            emax = jax.ops.segment_max(e, dst, num_segments=ni)
            emax = jnp.where(jnp.isfinite(emax), emax, 0.0)
            ex = jnp.exp(e - emax[dst])
            den = jax.ops.segment_sum(ex, dst, num_segments=ni)
            alpha = ex / (den[dst] + 1e-9)
            w = eft.reshape(en, HEADS, DH) * alpha[:, :, None]
            agg = jax.ops.segment_sum(w.reshape(en, HD), dst, num_segments=ni)
            outs.append(jnp.where(agg > 0, agg, jnp.expm1(agg)))
        s = jnp.stack([jnp.mean(jnp.tanh(o @ Wb[i] + bb[i]) @ qb[i]) for o in outs])
        beta = jax.nn.softmax(s)
        his.append(beta[0] * outs[0] + beta[1] * outs[1])
        beta_last = beta
    return his, beta_last


def kernel(feat0, feat1, adj, emb, enc_W, enc_b, W_ac, Wh0, av0, Wb0, bb0, qb0,
           fcW0, fcb0, Wh1, av1, Wb1, bb1, qb1, fcW1, fcb1, type_mask,
           feat_keep_idx, feat_drop_idx, mp0_idx, mp1_idx,
           target_node_indices, epoch, flag):
    t0 = _fc(feat0, enc_W[0], enc_b[0], act='elu', mb=400)

    k1p = 2816
    ed1 = emb[feat_drop_idx]
    es1T = jnp.pad(emb[feat_keep_idx], ((0, k1p - 2800), (0, 0))).T
    fs1 = jnp.pad(t0[feat_keep_idx], ((0, k1p - 2800), (0, 0)))
    adj1 = jnp.pad(adj[feat_drop_idx][:, feat_keep_idx], ((0, 0), (0, k1p - 2800)))
    re_drop = _att(ed1, es1T, fs1, adj1, W_ac, mb=400)
    d = t0[feat_drop_idx] - re_drop
    loss_ac = jnp.mean(d * d)

    k2p = 4096
    es2T = jnp.pad(emb[:N0], ((0, k2p - N0), (0, 0))).T
    fs2 = jnp.pad(t0, ((0, k2p - N0), (0, 0)))
    adj2 = jnp.pad(adj[N0:, :N0], ((0, 0), (0, k2p - N0)))
    feat_ac = _att(emb[N0:], es2T, fs2, adj2, W_ac, mb=400)

    tf = jnp.concatenate([t0, feat_ac], axis=0)

    his1, _ = _layer(tf, mp0_idx, mp1_idx, Wh0, av0, Wb0, bb0, qb0)
    h = _fc(jnp.concatenate(his1, axis=0), fcW0, fcb0, act='elu', mb=600)

    his2, beta2 = _layer(h, mp0_idx, mp1_idx, Wh1, av1, Wb1, bb1, qb1)
    ht = his2[0][target_node_indices]
    logits = _fc(ht, fcW1, fcb1, act=None, mb=1000)
    return logits, ht, beta2, loss_ac, jnp.zeros((), jnp.float32)


# speedup vs baseline: 4.5174x; 1.1303x over previous
import functools

import jax
import jax.numpy as jnp
from jax.experimental import pallas as pl

N0, N1, DIN, DH, HEADS, DEMB, ATT, OUT = 4000, 2000, 128, 128, 4, 64, 64, 8
HD = HEADS * DH


def _fc_kernel(x_ref, w_ref, b_ref, o_ref, *, act):
    y = jnp.dot(x_ref[:], w_ref[:], preferred_element_type=jnp.float32) + b_ref[:]
    if act == 'elu':
        y = jnp.where(y > 0, y, jnp.exp(y) - 1.0)
    o_ref[:] = y


def _fc(x, w, b, act, mb):
    m, din = x.shape
    dout = w.shape[1]
    return pl.pallas_call(
        functools.partial(_fc_kernel, act=act),
        grid=(m // mb,),
        in_specs=[pl.BlockSpec((mb, din), lambda i: (i, 0)),
                  pl.BlockSpec((din, dout), lambda i: (0, 0)),
                  pl.BlockSpec((1, dout), lambda i: (0, 0))],
        out_specs=pl.BlockSpec((mb, dout), lambda i: (i, 0)),
        out_shape=jax.ShapeDtypeStruct((m, dout), jnp.float32),
    )(x, w, b.reshape(1, -1))


def _att_kernel(ed_ref, esT_ref, fs_ref, adj_ref, w_ref, o_ref):
    ed = ed_ref[:]
    adj = adj_ref[:]
    esT = esT_ref[:]
    fs = fs_ref[:]
    acc = None
    for h in range(HEADS):
        w = w_ref[h]
        hd = jnp.dot(ed, w, preferred_element_type=jnp.float32)
        hd2 = jax.lax.dot_general(hd, w, (((1,), (1,)), ((), ())),
                                  preferred_element_type=jnp.float32)
        e = jnp.dot(hd2, esT, preferred_element_type=jnp.float32)
        e = jnp.where(e >= 0, e, 0.2 * e)
        e = jnp.where(adj > 0.5, e, -1e9)
        m = jnp.max(e, axis=1, keepdims=True)
        ex = jnp.exp(e - m)
        a = ex / jnp.sum(ex, axis=1, keepdims=True)
        t = jnp.dot(a, fs, preferred_element_type=jnp.float32)
        acc = t if acc is None else acc + t
    r = acc * (1.0 / HEADS)
    o_ref[:] = jnp.where(r > 0, r, jnp.exp(r) - 1.0)


def _att(ed, esT, fs, adj, w_ac, mb):
    m = ed.shape[0]
    kp = esT.shape[1]
    return pl.pallas_call(
        _att_kernel,
        grid=(m // mb,),
        in_specs=[pl.BlockSpec((mb, DEMB), lambda i: (i, 0)),
                  pl.BlockSpec((DEMB, kp), lambda i: (0, 0)),
                  pl.BlockSpec((kp, DH), lambda i: (0, 0)),
                  pl.BlockSpec((mb, kp), lambda i: (i, 0)),
                  pl.BlockSpec((HEADS, DEMB, DEMB), lambda i: (0, 0, 0))],
        out_specs=pl.BlockSpec((mb, DH), lambda i: (i, 0)),
        out_shape=jax.ShapeDtypeStruct((m, DH), jnp.float32),
    )(ed, esT, fs, adj, w_ac)


def _edge_kernel(h_ref, wh_ref, wv_ref, eft_ref, e_ref):
    h = h_ref[:]
    eft_ref[:] = jnp.dot(h, wh_ref[0], preferred_element_type=jnp.float32)
    z = jnp.dot(h, wv_ref[0], preferred_element_type=jnp.float32)
    e_ref[:] = jnp.where(z >= 0, z, 0.2 * z)


def _edge(h_inst, whs, wvs, eb, per_path):
    n2 = h_inst.shape[0]
    k = per_path // eb
    return pl.pallas_call(
        _edge_kernel,
        grid=(n2 // eb,),
        in_specs=[pl.BlockSpec((eb, DH), lambda i: (i, 0)),
                  pl.BlockSpec((1, DH, HD), lambda i, k=k: (i // k, 0, 0)),
                  pl.BlockSpec((1, DH, HEADS), lambda i, k=k: (i // k, 0, 0))],
        out_specs=[pl.BlockSpec((eb, HD), lambda i: (i, 0)),
                   pl.BlockSpec((eb, HEADS), lambda i: (i, 0))],
        out_shape=[jax.ShapeDtypeStruct((n2, HD), jnp.float32),
                   jax.ShapeDtypeStruct((n2, HEADS), jnp.float32)],
    )(h_inst, whs, wvs)


def _layer(tf, mp0, mp1, Wh, av, Wb, bb, qb, need_hi):
    his = []
    beta_last = None
    for i, (mp, ni, off) in enumerate([(mp0, N0, 0), (mp1, N1, N0)]):
        part = jax.lax.dynamic_slice_in_dim(tf, off, ni, 0)
        en = mp.shape[1]
        idx2 = mp.reshape(2 * en, 3)
        dst2 = idx2[:, 0] + jnp.where(jnp.arange(2 * en) < en, 0, ni)
        g = jnp.take(part, idx2.reshape(-1), axis=0).reshape(2 * en, 3, DH)
        h_inst = jnp.mean(g, axis=1)
        wvs = jnp.einsum('pdhk,phk->pdh', Wh[i].reshape(2, DH, HEADS, DH), av[i])
        eft, e = _edge(h_inst, Wh[i], wvs, eb=2000, per_path=en)
        ex = jnp.exp(e - jnp.max(e))
        num = (eft.reshape(-1, HEADS, DH) * ex[:, :, None]).reshape(-1, HD)
        payload = jnp.concatenate([num, ex], axis=1)
        seg = jax.ops.segment_sum(payload, dst2, num_segments=2 * ni)
        den = seg[:, HD:]
        agg = (seg[:, :HD].reshape(-1, HEADS, DH)
               / (den[:, :, None] + 1e-30)).reshape(-1, HD)
        o = jnp.where(agg > 0, agg, jnp.expm1(agg))
        outs = [o[:ni], o[ni:]]
        s = jnp.stack([jnp.mean(jnp.tanh(o_ @ Wb[i] + bb[i]) @ qb[i]) for o_ in outs])
        beta = jax.nn.softmax(s)
        if need_hi[i]:
            his.append(beta[0] * outs[0] + beta[1] * outs[1])
        else:
            his.append(None)
        beta_last = beta
    return his, beta_last


def kernel(feat0, feat1, adj, emb, enc_W, enc_b, W_ac, Wh0, av0, Wb0, bb0, qb0,
           fcW0, fcb0, Wh1, av1, Wb1, bb1, qb1, fcW1, fcb1, type_mask,
           feat_keep_idx, feat_drop_idx, mp0_idx, mp1_idx,
           target_node_indices, epoch, flag):
    t0 = _fc(feat0, enc_W[0], enc_b[0], act='elu', mb=400)

    k1p = 2816
    ed1 = emb[feat_drop_idx]
    es1T = jnp.pad(emb[feat_keep_idx], ((0, k1p - 2800), (0, 0))).T
    fs1 = jnp.pad(t0[feat_keep_idx], ((0, k1p - 2800), (0, 0)))
    adj1 = jnp.pad(adj[feat_drop_idx][:, feat_keep_idx], ((0, 0), (0, k1p - 2800)))
    re_drop = _att(ed1, es1T, fs1, adj1, W_ac, mb=400)
    d = t0[feat_drop_idx] - re_drop
    loss_ac = jnp.mean(d * d)

    k2p = 4096
    es2T = jnp.pad(emb[:N0], ((0, k2p - N0), (0, 0))).T
    fs2 = jnp.pad(t0, ((0, k2p - N0), (0, 0)))
    adj2 = jnp.pad(adj[N0:, :N0], ((0, 0), (0, k2p - N0)))
    feat_ac = _att(emb[N0:], es2T, fs2, adj2, W_ac, mb=400)

    tf = jnp.concatenate([t0, feat_ac], axis=0)

    his1, _ = _layer(tf, mp0_idx, mp1_idx, Wh0, av0, Wb0, bb0, qb0,
                     need_hi=(True, True))
    h = _fc(jnp.concatenate(his1, axis=0), fcW0, fcb0, act='elu', mb=600)

    his2, beta2 = _layer(h, mp0_idx, mp1_idx, Wh1, av1, Wb1, bb1, qb1,
                         need_hi=(True, False))
    ht = his2[0][target_node_indices]
    logits = _fc(ht, fcW1, fcb1, act=None, mb=1000)
    return logits, ht, beta2, loss_ac, jnp.zeros((), jnp.float32)


# speedup vs baseline: 5.5541x; 1.2295x over previous
import functools

import jax
import jax.numpy as jnp
from jax.experimental import pallas as pl

N0, N1, DIN, DH, HEADS, DEMB, ATT, OUT = 4000, 2000, 128, 128, 4, 64, 64, 8
HD = HEADS * DH


def _fc_kernel(x_ref, w_ref, b_ref, o_ref, *, act):
    y = jnp.dot(x_ref[:], w_ref[:], preferred_element_type=jnp.float32) + b_ref[:]
    if act == 'elu':
        y = jnp.where(y > 0, y, jnp.exp(y) - 1.0)
    o_ref[:] = y


def _fc(x, w, b, act, mb):
    m, din = x.shape
    dout = w.shape[1]
    return pl.pallas_call(
        functools.partial(_fc_kernel, act=act),
        grid=(m // mb,),
        in_specs=[pl.BlockSpec((mb, din), lambda i: (i, 0)),
                  pl.BlockSpec((din, dout), lambda i: (0, 0)),
                  pl.BlockSpec((1, dout), lambda i: (0, 0))],
        out_specs=pl.BlockSpec((mb, dout), lambda i: (i, 0)),
        out_shape=jax.ShapeDtypeStruct((m, dout), jnp.float32),
    )(x, w, b.reshape(1, -1))


def _att_kernel(ed_ref, esT_ref, fs_ref, adj_ref, w_ref, o_ref):
    ed = ed_ref[:]
    adj = adj_ref[:]
    esT = esT_ref[:]
    fs = fs_ref[:]
    acc = None
    for h in range(HEADS):
        w = w_ref[h]
        hd = jnp.dot(ed, w, preferred_element_type=jnp.float32)
        hd2 = jax.lax.dot_general(hd, w, (((1,), (1,)), ((), ())),
                                  preferred_element_type=jnp.float32)
        e = jnp.dot(hd2, esT, preferred_element_type=jnp.float32)
        e = jnp.where(e >= 0, e, 0.2 * e)
        e = jnp.where(adj > 0.5, e, -1e9)
        m = jnp.max(e, axis=1, keepdims=True)
        ex = jnp.exp(e - m)
        a = ex / jnp.sum(ex, axis=1, keepdims=True)
        t = jnp.dot(a, fs, preferred_element_type=jnp.float32)
        acc = t if acc is None else acc + t
    r = acc * (1.0 / HEADS)
    o_ref[:] = jnp.where(r > 0, r, jnp.exp(r) - 1.0)


def _att(ed, esT, fs, adj, w_ac, mb):
    m = ed.shape[0]
    kp = esT.shape[1]
    return pl.pallas_call(
        _att_kernel,
        grid=(m // mb,),
        in_specs=[pl.BlockSpec((mb, DEMB), lambda i: (i, 0)),
                  pl.BlockSpec((DEMB, kp), lambda i: (0, 0)),
                  pl.BlockSpec((kp, DH), lambda i: (0, 0)),
                  pl.BlockSpec((mb, kp), lambda i: (i, 0)),
                  pl.BlockSpec((HEADS, DEMB, DEMB), lambda i: (0, 0, 0))],
        out_specs=pl.BlockSpec((mb, DH), lambda i: (i, 0)),
        out_shape=jax.ShapeDtypeStruct((m, DH), jnp.float32),
    )(ed, esT, fs, adj, w_ac)


def _edge_kernel(h_ref, wh_ref, wv_ref, pay_ref):
    h = h_ref[:]
    z = jnp.dot(h, wh_ref[0], preferred_element_type=jnp.float32)
    zl = jnp.dot(h, wv_ref[0], preferred_element_type=jnp.float32)
    ex = jnp.exp(jnp.where(zl >= 0, zl, 0.2 * zl))
    for hh in range(HEADS):
        pay_ref[:, hh * DH:(hh + 1) * DH] = z[:, hh * DH:(hh + 1) * DH] * ex[:, hh:hh + 1]
    pay_ref[:, HD:] = ex


def _edge(h_inst, whs, wvs, eb, per_path):
    n2 = h_inst.shape[0]
    k = per_path // eb
    return pl.pallas_call(
        _edge_kernel,
        grid=(n2 // eb,),
        in_specs=[pl.BlockSpec((eb, DH), lambda i: (i, 0)),
                  pl.BlockSpec((1, DH, HD), lambda i, k=k: (i // k, 0, 0)),
                  pl.BlockSpec((1, DH, HEADS), lambda i, k=k: (i // k, 0, 0))],
        out_specs=pl.BlockSpec((eb, HD + HEADS), lambda i: (i, 0)),
        out_shape=jax.ShapeDtypeStruct((n2, HD + HEADS), jnp.float32),
    )(h_inst, whs, wvs)


def _layer(tf, mp0, mp1, Wh, av, Wb, bb, qb, need_hi):
    his = []
    beta_last = None
    for i, (mp, ni, off) in enumerate([(mp0, N0, 0), (mp1, N1, N0)]):
        part = jax.lax.dynamic_slice_in_dim(tf, off, ni, 0)
        en = mp.shape[1]
        idx2 = mp.reshape(2 * en, 3)
        dst2 = idx2[:, 0] + jnp.where(jnp.arange(2 * en) < en, 0, ni)
        g = jnp.take(part, idx2.reshape(-1), axis=0).reshape(2 * en, 3, DH)
        h_inst = jnp.mean(g, axis=1)
        wvs = jnp.einsum('pdhk,phk->pdh', Wh[i].reshape(2, DH, HEADS, DH), av[i])
        payload = _edge(h_inst, Wh[i], wvs, eb=2000, per_path=en)
        seg = jax.ops.segment_sum(payload, dst2, num_segments=2 * ni)
        den = seg[:, HD:]
        agg = (seg[:, :HD].reshape(-1, HEADS, DH)
               / (den[:, :, None] + 1e-30)).reshape(-1, HD)
        o = jnp.where(agg > 0, agg, jnp.expm1(agg))
        outs = [o[:ni], o[ni:]]
        s = jnp.stack([jnp.mean(jnp.tanh(o_ @ Wb[i] + bb[i]) @ qb[i]) for o_ in outs])
        beta = jax.nn.softmax(s)
        if need_hi[i]:
            his.append(beta[0] * outs[0] + beta[1] * outs[1])
        else:
            his.append(None)
        beta_last = beta
    return his, beta_last


def kernel(feat0, feat1, adj, emb, enc_W, enc_b, W_ac, Wh0, av0, Wb0, bb0, qb0,
           fcW0, fcb0, Wh1, av1, Wb1, bb1, qb1, fcW1, fcb1, type_mask,
           feat_keep_idx, feat_drop_idx, mp0_idx, mp1_idx,
           target_node_indices, epoch, flag):
    t0 = _fc(feat0, enc_W[0], enc_b[0], act='elu', mb=400)

    k1p = 2816
    ed1 = emb[feat_drop_idx]
    es1T = jnp.pad(emb[feat_keep_idx], ((0, k1p - 2800), (0, 0))).T
    fs1 = jnp.pad(t0[feat_keep_idx], ((0, k1p - 2800), (0, 0)))
    adj1 = jnp.pad(adj[feat_drop_idx][:, feat_keep_idx], ((0, 0), (0, k1p - 2800)))
    re_drop = _att(ed1, es1T, fs1, adj1, W_ac, mb=400)
    d = t0[feat_drop_idx] - re_drop
    loss_ac = jnp.mean(d * d)

    k2p = 4096
    es2T = jnp.pad(emb[:N0], ((0, k2p - N0), (0, 0))).T
    fs2 = jnp.pad(t0, ((0, k2p - N0), (0, 0)))
    adj2 = jnp.pad(adj[N0:, :N0], ((0, 0), (0, k2p - N0)))
    feat_ac = _att(emb[N0:], es2T, fs2, adj2, W_ac, mb=400)

    tf = jnp.concatenate([t0, feat_ac], axis=0)

    his1, _ = _layer(tf, mp0_idx, mp1_idx, Wh0, av0, Wb0, bb0, qb0,
                     need_hi=(True, True))
    h = _fc(jnp.concatenate(his1, axis=0), fcW0, fcb0, act='elu', mb=600)

    his2, beta2 = _layer(h, mp0_idx, mp1_idx, Wh1, av1, Wb1, bb1, qb1,
                         need_hi=(True, False))
    ht = his2[0][target_node_indices]
    logits = _fc(ht, fcW1, fcb1, act=None, mb=1000)
    return logits, ht, beta2, loss_ac, jnp.zeros((), jnp.float32)
